# baseline (speedup 1.0000x reference)
import functools

import jax
import jax.numpy as jnp
from jax import lax
from jax.experimental import pallas as pl
from jax.experimental.pallas import tpu as pltpu
from jax.experimental.pallas import tpu_sc as plsc

NC = 2
NS = 16
NW = NC * NS
L = 16


@functools.lru_cache(maxsize=None)
def _gather_kernel(W, V, B):
    rows_per_w = 2 * W // NW
    assert W % (NW // 2) == 0 and B % L == 0

    mesh = plsc.VectorSubcoreMesh(core_axis_name="c", subcore_axis_name="s")

    @functools.partial(
        pl.kernel,
        mesh=mesh,
        out_type=(
            jax.ShapeDtypeStruct((W, B), jnp.float32),
            jax.ShapeDtypeStruct((W, B), jnp.float32),
        ),
        scratch_types=[
            pltpu.VMEM((V,), jnp.float32),
            pltpu.VMEM((B,), jnp.int32),
            pltpu.VMEM((B // 2,), jnp.float32),
        ],
        compiler_params=pltpu.CompilerParams(
            use_tc_tiling_on_sc=True, needs_layout_passes=False
        ),
    )
    def k(scales_t, shifts_t, lab_hbm, gs_out, gh_out, row_v, idx_v, out_v):
        wid = lax.axis_index("s") * NC + lax.axis_index("c")
        H = B // 2
        UNROLL = 16
        pltpu.sync_copy(lab_hbm, idx_v)

        def do_rows(src, dst):
            for t in range(rows_per_w):
                w = wid * rows_per_w + t
                pltpu.sync_copy(src.at[w], row_v)
                for half in range(2):

                    def body(i, carry, half=half):
                        for u in range(UNROLL):
                            j = i * UNROLL + u
                            out_v[pl.ds(j * L, L)] = plsc.load_gather(
                                row_v, [idx_v[pl.ds(half * H + j * L, L)]]
                            )
                        return carry

                    lax.fori_loop(0, H // (L * UNROLL), body, 0)
                    pltpu.sync_copy(out_v, dst.at[w, pl.ds(half * H, H)])

        do_rows(scales_t, gs_out)
        do_rows(shifts_t, gh_out)

    return k


@functools.lru_cache(maxsize=None)
def _film_tc_kernel(S, W, B):
    BLK = 4096
    assert B % BLK == 0

    def body(x_ref, gs_ref, gh_ref, o_ref):
        o_ref[...] = x_ref[...] * gs_ref[...][None] + gh_ref[...][None]

    return pl.pallas_call(
        body,
        grid=(B // BLK, S),
        in_specs=[
            pl.BlockSpec((1, W, BLK), lambda i, s: (s, 0, i)),
            pl.BlockSpec((W, BLK), lambda i, s: (0, i)),
            pl.BlockSpec((W, BLK), lambda i, s: (0, i)),
        ],
        out_specs=pl.BlockSpec((1, W, BLK), lambda i, s: (s, 0, i)),
        out_shape=jax.ShapeDtypeStruct((S, W, B), jnp.float32),
    )


def kernel(x, task_labels, num_samples, scales, shifts):
    S, B, W = x.shape
    V = scales.shape[0]
    x_t = jnp.transpose(x, (0, 2, 1))
    scales_t = scales.T
    shifts_t = shifts.T
    labels = task_labels.astype(jnp.int32)
    gs_t, gh_t = _gather_kernel(W, V, B)(scales_t, shifts_t, labels)
    out_t = _film_tc_kernel(S, W, B)(x_t, gs_t, gh_t)
    return jnp.transpose(out_t, (0, 2, 1))


# speedup vs baseline: 1.4909x; 1.4909x over previous
import functools

import jax
import jax.numpy as jnp
from jax import lax
from jax.experimental import pallas as pl
from jax.experimental.pallas import tpu as pltpu
from jax.experimental.pallas import tpu_sc as plsc

NC = 2
NS = 16
NW = NC * NS
L = 16


@functools.lru_cache(maxsize=None)
def _gather_half_kernel(W, V, B, start, wh):
    assert wh == NW and B % L == 0

    mesh = plsc.VectorSubcoreMesh(core_axis_name="c", subcore_axis_name="s")

    @functools.partial(
        pl.kernel,
        mesh=mesh,
        out_type=(
            jax.ShapeDtypeStruct((wh, B), jnp.float32),
            jax.ShapeDtypeStruct((wh, B), jnp.float32),
        ),
        scratch_types=[
            pltpu.VMEM((V,), jnp.float32),
            pltpu.VMEM((B,), jnp.int32),
            pltpu.VMEM((B // 2,), jnp.float32),
        ],
        compiler_params=pltpu.CompilerParams(
            use_tc_tiling_on_sc=True, needs_layout_passes=False
        ),
    )
    def k(scales_t, shifts_t, lab_hbm, gs_out, gh_out, row_v, idx_v, out_v):
        wid = lax.axis_index("s") * NC + lax.axis_index("c")
        H = B // 2
        UNROLL = 8
        pltpu.sync_copy(lab_hbm, idx_v)

        def do_row(src, dst):
            w = start + wid
            pltpu.sync_copy(src.at[w], row_v)
            for half in range(2):

                def body(i, carry, half=half):
                    for u in range(UNROLL):
                        j = i * UNROLL + u
                        out_v[pl.ds(j * L, L)] = plsc.load_gather(
                            row_v, [idx_v[pl.ds(half * H + j * L, L)]]
                        )
                    return carry

                lax.fori_loop(0, H // (L * UNROLL), body, 0)
                pltpu.sync_copy(out_v, dst.at[wid, pl.ds(half * H, H)])

        do_row(scales_t, gs_out)
        do_row(shifts_t, gh_out)

    return k


@functools.lru_cache(maxsize=None)
def _film_tc_kernel(S, W, B, hidx, wh, first):
    BLK = 4096
    assert B % BLK == 0

    def body(*refs):
        if first:
            x_ref, gs_ref, gh_ref, o_ref = refs
        else:
            _, x_ref, gs_ref, gh_ref, o_ref = refs
        o_ref[...] = x_ref[...] * gs_ref[...][None] + gh_ref[...][None]

    xw_spec = pl.BlockSpec((S, wh, BLK), lambda i: (0, hidx, i))
    g_spec = pl.BlockSpec((wh, BLK), lambda i: (0, i))
    in_specs = [xw_spec, g_spec, g_spec]
    kwargs = {}
    if not first:
        in_specs = [pl.BlockSpec(memory_space=pltpu.MemorySpace.HBM)] + in_specs
        kwargs["input_output_aliases"] = {0: 0}

    return pl.pallas_call(
        body,
        grid=(B // BLK,),
        in_specs=in_specs,
        out_specs=pl.BlockSpec((S, wh, BLK), lambda i: (0, hidx, i)),
        out_shape=jax.ShapeDtypeStruct((S, W, B), jnp.float32),
        **kwargs,
    )


def kernel(x, task_labels, num_samples, scales, shifts):
    S, B, W = x.shape
    V = scales.shape[0]
    wh = W // 2
    x_t = jnp.transpose(x, (0, 2, 1))
    scales_t = scales.T
    shifts_t = shifts.T
    labels = task_labels.astype(jnp.int32)
    gs0, gh0 = _gather_half_kernel(W, V, B, 0, wh)(scales_t, shifts_t, labels)
    gs1, gh1 = _gather_half_kernel(W, V, B, wh, wh)(scales_t, shifts_t, labels)
    out0 = _film_tc_kernel(S, W, B, 0, wh, True)(x_t, gs0, gh0)
    out1 = _film_tc_kernel(S, W, B, 1, wh, False)(out0, x_t, gs1, gh1)
    return jnp.transpose(out1, (0, 2, 1))
